# baseline (speedup 1.0000x reference)
import jax
import jax.numpy as jnp
from jax import lax
from jax.experimental import pallas as pl
from jax.experimental.pallas import tpu as pltpu
from jax.experimental.pallas import tpu_sc as plsc

D = 64
N = 16384
VOCAB_SZ = 1000000
NC = 2
NS = 16
NW = NC * NS
PER_W = N // NW
ZCH = 16384
TPW = 4 * ZCH
WPAD = NS * TPW
VB = 8192


def _sc_scatter(ids_hbm, w_hbm, wout_hbm, wsum_hbm,
                idx_v, w_v, zero_v, pv, shared_w):
    cid = lax.axis_index("c")
    sid = lax.axis_index("s")
    wid = sid * NC + cid

    pltpu.sync_copy(ids_hbm.at[pl.ds(wid * 4, 4)], idx_v)
    pltpu.sync_copy(w_hbm.at[pl.ds(wid * 4, 4)], w_v)

    zero = jnp.zeros((16,), jnp.float32)

    def zbody(i, carry):
        zero_v[pl.ds(i * 16, 16)] = zero
        return carry

    lax.fori_loop(0, ZCH // 16, zbody, 0)
    for t in range(TPW // ZCH):
        pltpu.sync_copy(zero_v, shared_w.at[pl.ds(sid * TPW + t * ZCH, ZCH)])
    plsc.subcore_barrier()

    for k in range(4):
        pltpu.sync_copy(w_v.at[k], shared_w.at[idx_v.at[k]], add=True)

    wacc = zero
    for k in range(4):
        for l in range(8):
            wacc = wacc + w_v[k, pl.ds(l * 16, 16)]
    pv[pl.ds(0, 16)] = wacc
    for k in range(1, 8):
        pv[pl.ds(k * 16, 16)] = zero
    pltpu.sync_copy(pv, wsum_hbm.at[wid])

    plsc.subcore_barrier()

    for t in range(TPW // ZCH):
        off = sid * TPW + t * ZCH
        pltpu.sync_copy(shared_w.at[pl.ds(off, ZCH)],
                        wout_hbm.at[cid, pl.ds(off, ZCH)])


NBLK = (VOCAB_SZ + VB - 1) // VB


def _tc_matvec(tt_ref, w_ref, out_ref):
    @pl.when(pl.program_id(0) == 0)
    def _():
        out_ref[...] = jnp.zeros_like(out_ref)

    tt = tt_ref[...]
    v0 = pl.program_id(0) * VB

    def masked(t):
        vpos = v0 + lax.broadcasted_iota(jnp.int32, (D, VB), 1)
        return jnp.where(vpos < VOCAB_SZ, t, 0.0)

    tt = lax.cond(pl.program_id(0) == NBLK - 1, masked, lambda t: t, tt)
    ws = w_ref[0, :] + w_ref[1, :]
    prod = (tt * ws[None, :]).reshape(D, VB // 128, 128)
    out_ref[...] += jnp.sum(prod, axis=1)


def kernel(ids, weights, table):
    ids_r = ids.astype(jnp.int32).reshape(NW * 4, 128)
    w_r = weights.reshape(NW * 4, 128)
    table_t = table.T

    mesh = plsc.VectorSubcoreMesh(core_axis_name="c", subcore_axis_name="s")
    w_dense, w_parts = pl.kernel(
        _sc_scatter,
        mesh=mesh,
        out_type=[
            jax.ShapeDtypeStruct((NC, WPAD), jnp.float32),
            jax.ShapeDtypeStruct((NW, 128), jnp.float32),
        ],
        scratch_types=[
            pltpu.VMEM((4, 128), jnp.int32),
            pltpu.VMEM((4, 128), jnp.float32),
            pltpu.VMEM((ZCH,), jnp.float32),
            pltpu.VMEM((128,), jnp.float32),
            pltpu.VMEM_SHARED((WPAD,), jnp.float32),
        ],
    )(ids_r, w_r)

    out = pl.pallas_call(
        _tc_matvec,
        grid=(NBLK,),
        in_specs=[
            pl.BlockSpec((D, VB), lambda i: (0, i)),
            pl.BlockSpec((NC, VB), lambda i: (0, i)),
        ],
        out_specs=pl.BlockSpec((D, 128), lambda i: (0, 0)),
        out_shape=jax.ShapeDtypeStruct((D, 128), jnp.float32),
        compiler_params=pltpu.CompilerParams(
            dimension_semantics=("arbitrary",)),
    )(table_t, w_dense)

    wsum = w_parts[:, :16].sum()
    return out.sum(axis=1) / wsum


# speedup vs baseline: 1.0846x; 1.0846x over previous
import jax
import jax.numpy as jnp
from jax import lax
from jax.experimental import pallas as pl
from jax.experimental.pallas import tpu as pltpu
from jax.experimental.pallas import tpu_sc as plsc

D = 64
N = 16384
VOCAB_SZ = 1000000
NC = 2
NS = 16
NW = NC * NS
PER_W = N // NW
ZCH = 16384
TPW = 4 * ZCH
WPAD = NS * TPW
VB = 8192


def _sc_scatter(ids_hbm, w_hbm, wout_hbm, wsum_hbm,
                idx_v, w_v, zero_v, pv, shared_w):
    cid = lax.axis_index("c")
    sid = lax.axis_index("s")
    wid = sid * NC + cid

    pltpu.sync_copy(ids_hbm.at[pl.ds(wid * 4, 4)], idx_v)
    pltpu.sync_copy(w_hbm.at[pl.ds(wid * 4, 4)], w_v)

    zero = jnp.zeros((16,), jnp.float32)

    def zbody(i, carry):
        zero_v[pl.ds(i * 16, 16)] = zero
        return carry

    lax.fori_loop(0, ZCH // 16, zbody, 0)
    for t in range(TPW // ZCH):
        pltpu.sync_copy(zero_v, shared_w.at[pl.ds(sid * TPW + t * ZCH, ZCH)])
    plsc.subcore_barrier()

    for k in range(4):
        pltpu.sync_copy(w_v.at[k], shared_w.at[idx_v.at[k]], add=True)

    wacc = zero
    for k in range(4):
        for l in range(8):
            wacc = wacc + w_v[k, pl.ds(l * 16, 16)]
    pv[pl.ds(0, 16)] = wacc
    for k in range(1, 8):
        pv[pl.ds(k * 16, 16)] = zero
    pltpu.sync_copy(pv, wsum_hbm.at[wid])

    plsc.subcore_barrier()

    for t in range(TPW // ZCH):
        off = sid * TPW + t * ZCH
        pltpu.sync_copy(shared_w.at[pl.ds(off, ZCH)],
                        wout_hbm.at[cid, pl.ds(off, ZCH)])


NBLK = (VOCAB_SZ + VB - 1) // VB


def _tc_matvec(tt_ref, w_ref, out_ref):
    @pl.when(pl.program_id(0) == 0)
    def _():
        out_ref[...] = jnp.zeros_like(out_ref)

    tt = tt_ref[...]
    v0 = pl.program_id(0) * VB

    def masked(t):
        vpos = v0 + lax.broadcasted_iota(jnp.int32, (D, VB), 1)
        return jnp.where(vpos < VOCAB_SZ, t, 0.0)

    tt = lax.cond(pl.program_id(0) == NBLK - 1, masked, lambda t: t, tt)
    ws = w_ref[0, :] + w_ref[1, :]
    out_ref[...] += jnp.sum(tt * ws[None, :], axis=1)[None, :]


def kernel(ids, weights, table):
    ids_r = ids.astype(jnp.int32).reshape(NW * 4, 128)
    w_r = weights.reshape(NW * 4, 128)
    table_t = table.T

    mesh = plsc.VectorSubcoreMesh(core_axis_name="c", subcore_axis_name="s")
    w_dense, w_parts = pl.kernel(
        _sc_scatter,
        mesh=mesh,
        out_type=[
            jax.ShapeDtypeStruct((NC, WPAD), jnp.float32),
            jax.ShapeDtypeStruct((NW, 128), jnp.float32),
        ],
        scratch_types=[
            pltpu.VMEM((4, 128), jnp.int32),
            pltpu.VMEM((4, 128), jnp.float32),
            pltpu.VMEM((ZCH,), jnp.float32),
            pltpu.VMEM((128,), jnp.float32),
            pltpu.VMEM_SHARED((WPAD,), jnp.float32),
        ],
    )(ids_r, w_r)

    out = pl.pallas_call(
        _tc_matvec,
        grid=(NBLK,),
        in_specs=[
            pl.BlockSpec((D, VB), lambda i: (0, i)),
            pl.BlockSpec((NC, VB), lambda i: (0, i)),
        ],
        out_specs=pl.BlockSpec((1, D), lambda i: (0, 0)),
        out_shape=jax.ShapeDtypeStruct((1, D), jnp.float32),
        compiler_params=pltpu.CompilerParams(
            dimension_semantics=("arbitrary",)),
    )(table_t, w_dense)

    wsum = w_parts[:, :16].sum()
    return out[0] / wsum


# speedup vs baseline: 1.3069x; 1.2050x over previous
import jax
import jax.numpy as jnp
from jax import lax
from jax.experimental import pallas as pl
from jax.experimental.pallas import tpu as pltpu
from jax.experimental.pallas import tpu_sc as plsc

D = 64
N = 16384
VOCAB_SZ = 1000000
NC = 2
NS = 16
NW = NC * NS
PER_W = N // NW
ZCH = 16384
TPW = 4 * ZCH
WPAD = NS * TPW
VB = 8192


def _sc_scatter(ids_hbm, w_hbm, wout_hbm, wsum_hbm,
                idx_v, w_v, zero_v, pv, shared_w):
    cid = lax.axis_index("c")
    sid = lax.axis_index("s")
    wid = sid * NC + cid

    pltpu.sync_copy(ids_hbm.at[pl.ds(wid * 4, 4)], idx_v)
    pltpu.sync_copy(w_hbm.at[pl.ds(wid * 4, 4)], w_v)

    zero = jnp.zeros((16,), jnp.float32)

    def zbody(i, carry):
        zero_v[pl.ds(i * 16, 16)] = zero
        return carry

    lax.fori_loop(0, ZCH // 16, zbody, 0)
    for t in range(TPW // ZCH):
        pltpu.sync_copy(zero_v, shared_w.at[pl.ds(sid * TPW + t * ZCH, ZCH)])
    plsc.subcore_barrier()

    for k in range(4):
        pltpu.sync_copy(w_v.at[k], shared_w.at[idx_v.at[k]], add=True)

    wacc = zero
    for k in range(4):
        for l in range(8):
            wacc = wacc + w_v[k, pl.ds(l * 16, 16)]
    pv[pl.ds(0, 16)] = wacc
    for k in range(1, 8):
        pv[pl.ds(k * 16, 16)] = zero
    pltpu.sync_copy(pv, wsum_hbm.at[wid])

    plsc.subcore_barrier()

    for t in range(TPW // ZCH):
        off = sid * TPW + t * ZCH
        pltpu.sync_copy(shared_w.at[pl.ds(off, ZCH)],
                        wout_hbm.at[cid, pl.ds(off, ZCH)])


NBLK = (VOCAB_SZ + VB - 1) // VB


def _tc_matvec(tt_ref, w_ref, out_ref):
    @pl.when(pl.program_id(0) == 0)
    def _():
        out_ref[...] = jnp.zeros_like(out_ref)

    v0 = pl.program_id(0) * VB
    vpos = v0 + lax.broadcasted_iota(jnp.int32, (D, VB), 1)
    tt = jnp.where(vpos < VOCAB_SZ, tt_ref[...], 0.0)
    ws = w_ref[0, :] + w_ref[1, :]
    out_ref[...] += jnp.sum(tt * ws[None, :], axis=1)[None, :]


def kernel(ids, weights, table):
    ids_r = ids.astype(jnp.int32).reshape(NW * 4, 128)
    w_r = weights.reshape(NW * 4, 128)
    table_t = table.T

    mesh = plsc.VectorSubcoreMesh(core_axis_name="c", subcore_axis_name="s")
    w_dense, w_parts = pl.kernel(
        _sc_scatter,
        mesh=mesh,
        out_type=[
            jax.ShapeDtypeStruct((NC, WPAD), jnp.float32),
            jax.ShapeDtypeStruct((NW, 128), jnp.float32),
        ],
        scratch_types=[
            pltpu.VMEM((4, 128), jnp.int32),
            pltpu.VMEM((4, 128), jnp.float32),
            pltpu.VMEM((ZCH,), jnp.float32),
            pltpu.VMEM((128,), jnp.float32),
            pltpu.VMEM_SHARED((WPAD,), jnp.float32),
        ],
    )(ids_r, w_r)

    out = pl.pallas_call(
        _tc_matvec,
        grid=(NBLK,),
        in_specs=[
            pl.BlockSpec((D, VB), lambda i: (0, i)),
            pl.BlockSpec((NC, VB), lambda i: (0, i)),
        ],
        out_specs=pl.BlockSpec((1, D), lambda i: (0, 0)),
        out_shape=jax.ShapeDtypeStruct((1, D), jnp.float32),
        compiler_params=pltpu.CompilerParams(
            dimension_semantics=("arbitrary",)),
    )(table_t, w_dense)

    wsum = w_parts[:, :16].sum()
    return out[0] / wsum


# speedup vs baseline: 1.3441x; 1.0285x over previous
import jax
import jax.numpy as jnp
from jax import lax
from jax.experimental import pallas as pl
from jax.experimental.pallas import tpu as pltpu
from jax.experimental.pallas import tpu_sc as plsc

D = 64
N = 16384
VOCAB_SZ = 1000000
NC = 2
NS = 16
NW = NC * NS
PER_W = N // NW
ZCH = 16384
TPW = 4 * ZCH
WPAD = NS * TPW
VB = 8192
VTAIL0 = 999424
TAIL = VOCAB_SZ - VTAIL0


def _sc_scatter(ids_hbm, w_hbm, table_hbm, wout_hbm, wsum_hbm,
                idx_v, w_v, zero_v, pv, tbuf, wtail_v, shared_w):
    cid = lax.axis_index("c")
    sid = lax.axis_index("s")
    wid = sid * NC + cid

    pltpu.sync_copy(ids_hbm.at[pl.ds(wid * 4, 4)], idx_v)
    pltpu.sync_copy(w_hbm.at[pl.ds(wid * 4, 4)], w_v)

    zero = jnp.zeros((16,), jnp.float32)

    def zbody(i, carry):
        zero_v[pl.ds(i * 16, 16)] = zero
        return carry

    lax.fori_loop(0, ZCH // 16, zbody, 0)
    for t in range(TPW // ZCH):
        pltpu.sync_copy(zero_v, shared_w.at[pl.ds(sid * TPW + t * ZCH, ZCH)])
    plsc.subcore_barrier()

    for k in range(4):
        pltpu.sync_copy(w_v.at[k], shared_w.at[idx_v.at[k]], add=True)

    wacc = zero
    for k in range(4):
        for l in range(8):
            wacc = wacc + w_v[k, pl.ds(l * 16, 16)]
    pv[pl.ds(0, 16)] = wacc
    for k in range(5, 8):
        pv[pl.ds(k * 16, 16)] = zero

    plsc.subcore_barrier()

    for t in range(TPW // ZCH):
        off = sid * TPW + t * ZCH
        pltpu.sync_copy(shared_w.at[pl.ds(off, ZCH)],
                        wout_hbm.at[cid, pl.ds(off, ZCH)])

    band = sid // 2
    pltpu.sync_copy(table_hbm.at[band, :, pl.ds(VTAIL0, TAIL)], tbuf)
    pltpu.sync_copy(shared_w.at[pl.ds(VTAIL0, TAIL)], wtail_v)
    for j in range(4):
        s_d = (sid % 2) * 4 + j

        def tdot(c, acc):
            return acc + tbuf[s_d, pl.ds(c * 16, 16)] * wtail_v[pl.ds(c * 16, 16)]

        accj = lax.fori_loop(0, TAIL // 16, tdot, zero)
        pv[pl.ds((1 + j) * 16, 16)] = accj
    pltpu.sync_copy(pv, wsum_hbm.at[wid])


NBLK = VTAIL0 // VB


def _tc_matvec(tt_ref, w_ref, out_ref):
    @pl.when(pl.program_id(0) == 0)
    def _():
        out_ref[...] = jnp.zeros_like(out_ref)

    tt = tt_ref[...]
    ws = w_ref[0, :] + w_ref[1, :]
    out_ref[...] += jnp.sum(tt * ws[None, :], axis=1)[None, :]


def kernel(ids, weights, table):
    ids_r = ids.astype(jnp.int32).reshape(NW * 4, 128)
    w_r = weights.reshape(NW * 4, 128)
    table_t = table.T

    table_3d = table_t.reshape(8, 8, VOCAB_SZ)

    mesh = plsc.VectorSubcoreMesh(core_axis_name="c", subcore_axis_name="s")
    w_dense, w_parts = pl.kernel(
        _sc_scatter,
        mesh=mesh,
        out_type=[
            jax.ShapeDtypeStruct((NC, WPAD), jnp.float32),
            jax.ShapeDtypeStruct((NW, 128), jnp.float32),
        ],
        scratch_types=[
            pltpu.VMEM((4, 128), jnp.int32),
            pltpu.VMEM((4, 128), jnp.float32),
            pltpu.VMEM((ZCH,), jnp.float32),
            pltpu.VMEM((128,), jnp.float32),
            pltpu.VMEM((8, TAIL), jnp.float32),
            pltpu.VMEM((TAIL,), jnp.float32),
            pltpu.VMEM_SHARED((WPAD,), jnp.float32),
        ],
    )(ids_r, w_r, table_3d)

    out = pl.pallas_call(
        _tc_matvec,
        grid=(NBLK,),
        in_specs=[
            pl.BlockSpec((D, VB), lambda i: (0, i)),
            pl.BlockSpec((NC, VB), lambda i: (0, i)),
        ],
        out_specs=pl.BlockSpec((1, D), lambda i: (0, 0)),
        out_shape=jax.ShapeDtypeStruct((1, D), jnp.float32),
        compiler_params=pltpu.CompilerParams(
            dimension_semantics=("arbitrary",)),
    )(table_t, w_dense)

    wsum = w_parts[:, :16].sum()
    tails = w_parts[:, 16:80].reshape(NS, NC, 4, 16).sum(axis=(1, 3))
    return (out[0] + tails.reshape(D)) / wsum
